# baseline (speedup 1.0000x reference)
import jax
import jax.numpy as jnp
from jax.experimental import pallas as pl

_NQ = 8
_CB = 2048
_D = 256
_DIN = 512
_EPS = 1e-05
_R = 256


def _rvq_body(x_ref, ecb_ref, ehiT_ref, emidT_ref, eloT_ref,
              e2T_ref, out_ref):
    resT = x_ref[...].T
    iota = jax.lax.broadcasted_iota(jnp.int32, (_CB, _R), 0)
    for i in range(_NQ):
        sc = jax.lax.dot_general(
            ecb_ref[i], resT.astype(jnp.bfloat16),
            (((1,), (0,)), ((), ())), preferred_element_type=jnp.float32)
        sc = e2T_ref[:, i:i + 1] - 2.0 * sc
        m = jnp.min(sc, axis=0, keepdims=True)
        idx = jnp.min(jnp.where(sc == m, iota, _CB), axis=0)
        out_ref[i, :] = idx
        if i < _NQ - 1:
            oh = (iota == idx[None, :]).astype(jnp.bfloat16)
            q = jax.lax.dot_general(
                ehiT_ref[i], oh,
                (((1,), (0,)), ((), ())), preferred_element_type=jnp.float32)
            q = q + jax.lax.dot_general(
                emidT_ref[i], oh,
                (((1,), (0,)), ((), ())), preferred_element_type=jnp.float32)
            q = q + jax.lax.dot_general(
                eloT_ref[i], oh,
                (((1,), (0,)), ((), ())), preferred_element_type=jnp.float32)
            resT = resT - q


def kernel(embeddings, input_proj_w, embed_sums, cluster_usages):
    b, _, t = embeddings.shape
    n = b * t
    embeds = [embed_sums[i] / jnp.clip(cluster_usages[i], _EPS, None)[:, None]
              for i in range(_NQ)]
    e2 = [jnp.sum(e * e, axis=1) for e in embeds]
    e2T = jnp.stack(e2, axis=1)
    embed = jnp.stack(embeds)
    ecb = embed.astype(jnp.bfloat16)

    def _chunk(v):
        b = jax.lax.bitcast_convert_type(v, jnp.uint32)
        c = jax.lax.bitcast_convert_type(b & jnp.uint32(0xFFFF0000),
                                         jnp.float32)
        return c, v - c

    ehi, rem = _chunk(embed)
    emid, rem = _chunk(rem)
    elo, _ = _chunk(rem)
    ehiT = jnp.transpose(ehi.astype(jnp.bfloat16), (0, 2, 1))
    emidT = jnp.transpose(emid.astype(jnp.bfloat16), (0, 2, 1))
    eloT = jnp.transpose(elo.astype(jnp.bfloat16), (0, 2, 1))

    x = jnp.einsum('oc,bct->bot', input_proj_w, embeddings)
    h = jnp.transpose(x, (0, 2, 1)).reshape(-1, _D)

    grid = (n // _R,)
    out = pl.pallas_call(
        _rvq_body,
        grid=grid,
        in_specs=[
            pl.BlockSpec((_R, _D), lambda r: (r, 0)),
            pl.BlockSpec((_NQ, _CB, _D), lambda r: (0, 0, 0)),
            pl.BlockSpec((_NQ, _D, _CB), lambda r: (0, 0, 0)),
            pl.BlockSpec((_NQ, _D, _CB), lambda r: (0, 0, 0)),
            pl.BlockSpec((_NQ, _D, _CB), lambda r: (0, 0, 0)),
            pl.BlockSpec((_CB, _NQ), lambda r: (0, 0)),
        ],
        out_specs=pl.BlockSpec((_NQ, _R), lambda r: (0, r)),
        out_shape=jax.ShapeDtypeStruct((_NQ, n), jnp.int32),
    )(h, ecb, ehiT, emidT, eloT, e2T)
    return out.reshape(_NQ, b, t)


# speedup vs baseline: 1.6139x; 1.6139x over previous
import jax
import jax.numpy as jnp
from jax.experimental import pallas as pl

_NQ = 8
_CB = 2048
_D = 256
_DIN = 512
_EPS = 1e-05
_R = 512


_RH = 256


def _rvq_body(x_ref, ecb_ref, ehiT_ref, emidT_ref, eloT_ref,
              e2T_ref, out_ref):
    res = [x_ref[0:_RH, :].T, x_ref[_RH:_R, :].T]
    iota = jax.lax.broadcasted_iota(jnp.int32, (_CB, _RH), 0)
    for i in range(_NQ):
        e2col = e2T_ref[:, i:i + 1]
        for s in range(2):
            sc = jax.lax.dot_general(
                ecb_ref[i], res[s].astype(jnp.bfloat16),
                (((1,), (0,)), ((), ())), preferred_element_type=jnp.float32)
            sc = e2col + sc
            m = jnp.min(sc, axis=0, keepdims=True)
            idx = jnp.min(jnp.where(sc == m, iota, _CB), axis=0)
            out_ref[i, s * _RH:(s + 1) * _RH] = idx
            if i < _NQ - 1:
                oh = (iota == idx[None, :]).astype(jnp.bfloat16)
                q = jax.lax.dot_general(
                    ehiT_ref[i], oh,
                    (((1,), (0,)), ((), ())), preferred_element_type=jnp.float32)
                q = q + jax.lax.dot_general(
                    emidT_ref[i], oh,
                    (((1,), (0,)), ((), ())), preferred_element_type=jnp.float32)
                q = q + jax.lax.dot_general(
                    eloT_ref[i], oh,
                    (((1,), (0,)), ((), ())), preferred_element_type=jnp.float32)
                res[s] = res[s] - q


def kernel(embeddings, input_proj_w, embed_sums, cluster_usages):
    b, _, t = embeddings.shape
    n = b * t
    embeds = [embed_sums[i] / jnp.clip(cluster_usages[i], _EPS, None)[:, None]
              for i in range(_NQ)]
    e2 = [jnp.sum(e * e, axis=1) for e in embeds]
    e2T = jnp.stack(e2, axis=1)
    embed = jnp.stack(embeds)
    ecb_bits = jax.lax.bitcast_convert_type(
        embed.astype(jnp.bfloat16), jnp.uint16)
    ecb = jax.lax.bitcast_convert_type(
        (ecb_bits ^ jnp.uint16(0x8000)) + jnp.uint16(0x0080), jnp.bfloat16)

    def _chunk(v):
        b = jax.lax.bitcast_convert_type(v, jnp.uint32)
        c = jax.lax.bitcast_convert_type(b & jnp.uint32(0xFFFF0000),
                                         jnp.float32)
        return c, v - c

    ehi, rem = _chunk(embed)
    emid, rem = _chunk(rem)
    elo, _ = _chunk(rem)
    ehiT = jnp.transpose(ehi.astype(jnp.bfloat16), (0, 2, 1))
    emidT = jnp.transpose(emid.astype(jnp.bfloat16), (0, 2, 1))
    eloT = jnp.transpose(elo.astype(jnp.bfloat16), (0, 2, 1))

    x = jnp.einsum('oc,bct->bot', input_proj_w, embeddings)
    h = jnp.transpose(x, (0, 2, 1)).reshape(-1, _D)

    grid = (n // _R,)
    out = pl.pallas_call(
        _rvq_body,
        grid=grid,
        in_specs=[
            pl.BlockSpec((_R, _D), lambda r: (r, 0)),
            pl.BlockSpec((_NQ, _CB, _D), lambda r: (0, 0, 0)),
            pl.BlockSpec((_NQ, _D, _CB), lambda r: (0, 0, 0)),
            pl.BlockSpec((_NQ, _D, _CB), lambda r: (0, 0, 0)),
            pl.BlockSpec((_NQ, _D, _CB), lambda r: (0, 0, 0)),
            pl.BlockSpec((_CB, _NQ), lambda r: (0, 0)),
        ],
        out_specs=pl.BlockSpec((_NQ, _R), lambda r: (0, r)),
        out_shape=jax.ShapeDtypeStruct((_NQ, n), jnp.int32),
    )(h, ecb, ehiT, emidT, eloT, e2T)
    return out.reshape(_NQ, b, t)


# speedup vs baseline: 1.6569x; 1.0266x over previous
import jax
import jax.numpy as jnp
from jax.experimental import pallas as pl

_NQ = 8
_CB = 2048
_D = 256
_DIN = 512
_EPS = 1e-05
_R = 512
_RH = 256


def _rvq_body(x_ref, ecb_ref, e3T_ref, e2T_ref, out_ref):
    res = [x_ref[0:_RH, :].T, x_ref[_RH:_R, :].T]
    iota = jax.lax.broadcasted_iota(jnp.int32, (_CB, _RH), 0)
    for i in range(_NQ):
        e2col = e2T_ref[:, i:i + 1]
        for s in range(2):
            sc = jax.lax.dot_general(
                ecb_ref[i], res[s].astype(jnp.bfloat16),
                (((1,), (0,)), ((), ())), preferred_element_type=jnp.float32)
            sc = e2col - 2.0 * sc
            m = jnp.min(sc, axis=0, keepdims=True)
            idx = jnp.min(jnp.where(sc == m, iota, _CB), axis=0)
            out_ref[i, s * _RH:(s + 1) * _RH] = idx
            if i < _NQ - 1:
                oh = (iota == idx[None, :]).astype(jnp.bfloat16)
                q3 = jax.lax.dot_general(
                    e3T_ref[i], oh,
                    (((1,), (0,)), ((), ())), preferred_element_type=jnp.float32)
                q = (q3[0:_D] + q3[_D:2 * _D]) + q3[2 * _D:3 * _D]
                res[s] = res[s] - q


def kernel(embeddings, input_proj_w, embed_sums, cluster_usages):
    b, _, t = embeddings.shape
    n = b * t
    embeds = [embed_sums[i] / jnp.clip(cluster_usages[i], _EPS, None)[:, None]
              for i in range(_NQ)]
    e2 = [jnp.sum(e * e, axis=1) for e in embeds]
    e2T = jnp.stack(e2, axis=1)
    embed = jnp.stack(embeds)
    ecb = embed.astype(jnp.bfloat16)

    def _chunk(v):
        bits = jax.lax.bitcast_convert_type(v, jnp.uint32)
        c = jax.lax.bitcast_convert_type(bits & jnp.uint32(0xFFFF0000),
                                         jnp.float32)
        return c, v - c

    ehi, rem = _chunk(embed)
    emid, rem = _chunk(rem)
    elo, _ = _chunk(rem)
    e3T = jnp.concatenate([
        jnp.transpose(ehi.astype(jnp.bfloat16), (0, 2, 1)),
        jnp.transpose(emid.astype(jnp.bfloat16), (0, 2, 1)),
        jnp.transpose(elo.astype(jnp.bfloat16), (0, 2, 1)),
    ], axis=1)

    x = jnp.einsum('oc,bct->bot', input_proj_w, embeddings)
    h = jnp.transpose(x, (0, 2, 1)).reshape(-1, _D)

    grid = (n // _R,)
    out = pl.pallas_call(
        _rvq_body,
        grid=grid,
        in_specs=[
            pl.BlockSpec((_R, _D), lambda r: (r, 0)),
            pl.BlockSpec((_NQ, _CB, _D), lambda r: (0, 0, 0)),
            pl.BlockSpec((_NQ, 3 * _D, _CB), lambda r: (0, 0, 0)),
            pl.BlockSpec((_CB, _NQ), lambda r: (0, 0)),
        ],
        out_specs=pl.BlockSpec((_NQ, _R), lambda r: (0, r)),
        out_shape=jax.ShapeDtypeStruct((_NQ, n), jnp.int32),
    )(h, ecb, e3T, e2T)
    return out.reshape(_NQ, b, t)


# speedup vs baseline: 2.0436x; 1.2334x over previous
import jax
import jax.numpy as jnp
from jax.experimental import pallas as pl

_NQ = 8
_CB = 2048
_D = 256
_DIN = 512
_EPS = 1e-05
_R = 512
_RH = 256


def _rvq_body(x_ref, ecb_ref, e3T_ref, e2T_ref, out_ref):
    res = [x_ref[0:_RH, :].T, x_ref[_RH:_R, :].T]
    iota = jax.lax.broadcasted_iota(jnp.int32, (_CB, _RH), 0)
    for i in range(_NQ):
        e2col = e2T_ref[:, i:i + 1]
        for s in range(2):
            sc = jax.lax.dot_general(
                ecb_ref[i], res[s].astype(jnp.bfloat16),
                (((1,), (0,)), ((), ())), preferred_element_type=jnp.float32)
            sc = e2col - 2.0 * sc
            idx = jnp.argmin(sc, axis=0)
            out_ref[i, s * _RH:(s + 1) * _RH] = idx
            if i < _NQ - 1:
                oh = (iota == idx[None, :]).astype(jnp.bfloat16)
                q3 = jax.lax.dot_general(
                    e3T_ref[i], oh,
                    (((1,), (0,)), ((), ())), preferred_element_type=jnp.float32)
                q = (q3[0:_D] + q3[_D:2 * _D]) + q3[2 * _D:3 * _D]
                res[s] = res[s] - q


def kernel(embeddings, input_proj_w, embed_sums, cluster_usages):
    b, _, t = embeddings.shape
    n = b * t
    embeds = [embed_sums[i] / jnp.clip(cluster_usages[i], _EPS, None)[:, None]
              for i in range(_NQ)]
    e2 = [jnp.sum(e * e, axis=1) for e in embeds]
    e2T = jnp.stack(e2, axis=1)
    embed = jnp.stack(embeds)
    ecb = embed.astype(jnp.bfloat16)

    def _chunk(v):
        bits = jax.lax.bitcast_convert_type(v, jnp.uint32)
        c = jax.lax.bitcast_convert_type(bits & jnp.uint32(0xFFFF0000),
                                         jnp.float32)
        return c, v - c

    ehi, rem = _chunk(embed)
    emid, rem = _chunk(rem)
    elo, _ = _chunk(rem)
    e3T = jnp.concatenate([
        jnp.transpose(ehi.astype(jnp.bfloat16), (0, 2, 1)),
        jnp.transpose(emid.astype(jnp.bfloat16), (0, 2, 1)),
        jnp.transpose(elo.astype(jnp.bfloat16), (0, 2, 1)),
    ], axis=1)

    x = jnp.einsum('oc,bct->bot', input_proj_w, embeddings)
    h = jnp.transpose(x, (0, 2, 1)).reshape(-1, _D)

    grid = (n // _R,)
    out = pl.pallas_call(
        _rvq_body,
        grid=grid,
        in_specs=[
            pl.BlockSpec((_R, _D), lambda r: (r, 0)),
            pl.BlockSpec((_NQ, _CB, _D), lambda r: (0, 0, 0)),
            pl.BlockSpec((_NQ, 3 * _D, _CB), lambda r: (0, 0, 0)),
            pl.BlockSpec((_CB, _NQ), lambda r: (0, 0)),
        ],
        out_specs=pl.BlockSpec((_NQ, _R), lambda r: (0, r)),
        out_shape=jax.ShapeDtypeStruct((_NQ, n), jnp.int32),
    )(h, ecb, e3T, e2T)
    return out.reshape(_NQ, b, t)
